# baseline (speedup 1.0000x reference)
import functools

import jax
import jax.numpy as jnp
from jax import lax
from jax.experimental import pallas as pl
from jax.experimental.pallas import tpu as pltpu
from jax.experimental.pallas import tpu_sc as plsc

NC = 2
NS = 16
L = 16
NW = NC * NS

B = 16384
D = 64
CHUNK = 128
CPW = B // NW
NCHUNK = CPW // CHUNK

NEG_SLOPE = 0.01


def _leaky_relu_inplace(buf):
    @pl.loop(0, CHUNK)
    def _(r):
        for c in range(D // L):
            slc = (pl.ds(r, 1), pl.ds(c * L, L))
            v = buf.at[*slc][...]
            buf.at[*slc][...] = jnp.maximum(v, v * NEG_SLOPE)


def kernel(all_id, table):
    idx3 = all_id.reshape(NW, NCHUNK, CHUNK)
    mesh = plsc.VectorSubcoreMesh(core_axis_name="c", subcore_axis_name="s")

    @functools.partial(
        pl.kernel,
        out_type=jax.ShapeDtypeStruct((B, D), jnp.float32),
        mesh=mesh,
        compiler_params=pltpu.CompilerParams(use_tc_tiling_on_sc=False),
        scratch_types=(
            [pltpu.VMEM((NCHUNK, CHUNK), jnp.int32)]
            + [pltpu.VMEM((CHUNK, D), jnp.float32) for _ in range(NCHUNK)]
            + [pltpu.SemaphoreType.DMA for _ in range(2 * NCHUNK)]
        ),
    )
    def k(idx_hbm, table_hbm, out_hbm, idx_v, *rest):
        bufs = rest[:NCHUNK]
        gsems = rest[NCHUNK:2 * NCHUNK]
        osems = rest[2 * NCHUNK:]

        wid = lax.axis_index("s") * NC + lax.axis_index("c")
        base = wid * CPW

        pltpu.sync_copy(idx_hbm.at[wid], idx_v)

        gathers = [
            pltpu.async_copy(table_hbm.at[idx_v.at[j]], bufs[j], gsems[j])
            for j in range(NCHUNK)
        ]
        outs = []
        for j in range(NCHUNK):
            gathers[j].wait()
            _leaky_relu_inplace(bufs[j])
            outs.append(
                pltpu.async_copy(
                    bufs[j], out_hbm.at[pl.ds(base + j * CHUNK, CHUNK)], osems[j]
                )
            )
        for o in outs:
            o.wait()

    return k(idx3, table)


# speedup vs baseline: 1.6982x; 1.6982x over previous
import functools

import jax
import jax.numpy as jnp
from jax import lax
from jax.experimental import pallas as pl
from jax.experimental.pallas import tpu as pltpu
from jax.experimental.pallas import tpu_sc as plsc

NC = 2
NS = 16
L = 16
NW = NC * NS

B = 16384
D = 64
CHUNK = 64
CPW = B // NW
NCHUNK = CPW // CHUNK

NEG_SLOPE = 0.01


def _leaky_relu_inplace(buf):
    @pl.loop(0, CHUNK)
    def _(r):
        for c in range(D // L):
            v = buf[r, pl.ds(c * L, L)]
            buf[r, pl.ds(c * L, L)] = jnp.maximum(v, v * NEG_SLOPE)


def kernel(all_id, table):
    idx2 = all_id.reshape(NW, CPW)
    mesh = plsc.VectorSubcoreMesh(core_axis_name="c", subcore_axis_name="s")

    @functools.partial(
        pl.kernel,
        out_type=jax.ShapeDtypeStruct((B, D), jnp.float32),
        mesh=mesh,
        compiler_params=pltpu.CompilerParams(needs_layout_passes=False),
        scratch_types=(
            [pltpu.VMEM((CPW,), jnp.int32)]
            + [pltpu.VMEM((CHUNK, D), jnp.float32) for _ in range(2)]
            + [pltpu.SemaphoreType.DMA for _ in range(4)]
        ),
    )
    def k(idx_hbm, table_hbm, out_hbm, idx_v, rbuf0, rbuf1,
          g0, g1, o0, o1):
        rbufs = (rbuf0, rbuf1)
        gsems = (g0, g1)
        osems = (o0, o1)

        wid = lax.axis_index("s") * NC + lax.axis_index("c")
        base = wid * CPW

        pltpu.sync_copy(idx_hbm.at[wid], idx_v)
        lanes = lax.iota(jnp.int32, L)

        def fire_chunk(j):
            buf = rbufs[j % 2]
            gsem = gsems[j % 2]

            @pl.loop(0, CHUNK // L)
            def _(g):
                vec = idx_v[pl.ds(j * CHUNK + g * L, L)]
                for r in range(L):
                    row = jnp.max(jnp.where(lanes == r, vec, -1))
                    pltpu.async_copy(
                        table_hbm.at[pl.ds(row, 1)],
                        buf.at[pl.ds(g * L + r, 1)],
                        gsem,
                    )

        def drain_chunk(j):
            pltpu.make_async_copy(
                table_hbm.at[pl.ds(0, CHUNK)], rbufs[j % 2], gsems[j % 2]
            ).wait()

        outs = [None] * NCHUNK
        fire_chunk(0)
        for j in range(NCHUNK):
            drain_chunk(j)
            if j + 1 < NCHUNK:
                if j >= 1:
                    outs[j - 1].wait()
                fire_chunk(j + 1)
            _leaky_relu_inplace(rbufs[j % 2])
            outs[j] = pltpu.async_copy(
                rbufs[j % 2],
                out_hbm.at[pl.ds(base + j * CHUNK, CHUNK)],
                osems[j % 2],
            )
        outs[NCHUNK - 2].wait()
        outs[NCHUNK - 1].wait()

    return k(idx2, table)


# speedup vs baseline: 2.7613x; 1.6260x over previous
import functools

import jax
import jax.numpy as jnp
from jax import lax
from jax.experimental import pallas as pl
from jax.experimental.pallas import tpu as pltpu
from jax.experimental.pallas import tpu_sc as plsc

NC = 2
NS = 16
L = 16
NW = NC * NS

B = 16384
D = 64
V = 1000000

SLABW = 512
NSLAB = V // SLABW + 1
TAILS = V // SLABW
TAILW = V - TAILS * SLABW
NT = (NSLAB + NW - 1) // NW
NPAIR = (NT + 1) // 2

HITMAX = 768
SSTRIP = 2048
NSTRIP = B // SSTRIP
OUTW = 128
RING = 256
DUMMY = B

NEG_SLOPE = 0.01


def kernel(all_id, table):
    table_t = table.T
    tail_t = jnp.pad(table_t[:, TAILS * SLABW:], ((0, 0), (0, 128 - TAILW)))
    mesh = plsc.VectorSubcoreMesh(core_axis_name="c", subcore_axis_name="s")

    @functools.partial(
        pl.kernel,
        out_type=jax.ShapeDtypeStruct((B + 8, OUTW), jnp.float32),
        mesh=mesh,
        compiler_params=pltpu.CompilerParams(needs_layout_passes=False),
        scratch_types=(
            [
                pltpu.VMEM((SSTRIP,), jnp.int32),
                pltpu.VMEM((SSTRIP,), jnp.int32),
                pltpu.VMEM((HITMAX + L,), jnp.int32),
                pltpu.VMEM((128 + L,), jnp.int32),
                pltpu.VMEM((D, SLABW), jnp.float32),
                pltpu.VMEM((D, SLABW), jnp.float32),
                pltpu.VMEM((RING, OUTW), jnp.float32),
                pltpu.VMEM((2, 128), jnp.int32),
                pltpu.SMEM((8,), jnp.int32),
            ]
            + [pltpu.SemaphoreType.DMA for _ in range(4)]
        ),
    )
    def k(idx_hbm, table_hbm, tail_hbm, out_hbm, strip0, strip1, hitbuf,
          slabhits, slab0, slab1, ring, oidx, cnt_s, ss0, ss1, sg0, sg1):
        strips = (strip0, strip1)
        ssems = (ss0, ss1)
        slabs = (slab0, slab1)
        gsems = (sg0, sg1)

        wid = lax.axis_index("s") * NC + lax.axis_index("c")
        lanes = lax.iota(jnp.int32, L)

        cnt_s[0] = 0
        cnt_s[1] = 0
        cnt_s[2] = 0

        @pl.loop(0, (HITMAX + L) // L)
        def _(g):
            hitbuf[pl.ds(g * L, L)] = jnp.full((L,), -1, jnp.int32)

        def fire_slab(t, tbuf):
            sid = wid + NW * t
            buf = slabs[tbuf]
            sem = gsems[tbuf]

            @pl.when(sid < TAILS)
            def _():
                pltpu.async_copy(
                    table_hbm.at[pl.ds(0, D), pl.ds(sid * SLABW, SLABW)],
                    buf, sem,
                )

            @pl.when(sid == TAILS)
            def _():
                pltpu.async_copy(
                    tail_hbm, buf.at[pl.ds(0, D), pl.ds(0, 128)], sem
                )

        def wait_slab(t, tbuf):
            sid = wid + NW * t
            buf = slabs[tbuf]
            sem = gsems[tbuf]

            @pl.when(sid < TAILS)
            def _():
                pltpu.make_async_copy(
                    table_hbm.at[pl.ds(0, D), pl.ds(0, SLABW)], buf, sem
                ).wait()

            @pl.when(sid == TAILS)
            def _():
                pltpu.make_async_copy(
                    tail_hbm, buf.at[pl.ds(0, D), pl.ds(0, 128)], sem
                ).wait()

        fire_slab(0, 0)
        fire_slab(1, 1)

        pltpu.async_copy(idx_hbm.at[pl.ds(0, SSTRIP)], strip0, ss0)
        for s in range(NSTRIP):
            if s + 1 < NSTRIP:
                pltpu.async_copy(
                    idx_hbm.at[pl.ds((s + 1) * SSTRIP, SSTRIP)],
                    strips[(s + 1) % 2], ssems[(s + 1) % 2],
                )
            pltpu.make_async_copy(
                idx_hbm.at[pl.ds(0, SSTRIP)], strips[s % 2], ssems[s % 2]
            ).wait()
            strip = strips[s % 2]

            @pl.loop(0, SSTRIP // L)
            def _(g, s=s, strip=strip):
                vec = strip[pl.ds(g * L, L)]
                mine = ((vec >> 9) & (NW - 1)) == wid
                kvec = s * SSTRIP + g * L + lanes
                pack = ((vec >> 14) << 23) | (kvec << 9) | (vec & (SLABW - 1))
                cnt = cnt_s[0]
                plsc.store_compressed(
                    hitbuf.at[pl.ds(cnt, L)], pack, mask=mine
                )
                cnt_s[0] = cnt + jnp.sum(jnp.where(mine, 1, 0))

        def flush_ring():
            fl = cnt_s[2]
            half = (fl >> 7) & 1
            pltpu.sync_copy(
                ring.at[pl.ds(fl & (RING - 1), 128)], out_hbm.at[oidx.at[half]]
            )
            cnt_s[2] = fl + 128

        def process_slab(t, tbuf):
            sid = wid + NW * t
            buf = slabs[tbuf]

            @pl.when(sid <= TAILS)
            def _():
                wait_slab(t, tbuf)
                cnt_s[3] = 0

                @pl.loop(0, HITMAX // L)
                def _(g):
                    e = hitbuf[pl.ds(g * L, L)]
                    m = (e >> 23) == t
                    sc = cnt_s[3]
                    plsc.store_compressed(
                        slabhits.at[pl.ds(sc, L)], e & 0x7FFFFF, mask=m
                    )
                    cnt_s[3] = sc + jnp.sum(jnp.where(m, 1, 0))

                sc = cnt_s[3]

                @pl.loop(0, (sc + L - 1) // L)
                def _(h):
                    oc = cnt_s[1]

                    @pl.when(oc - cnt_s[2] >= 128)
                    def _():
                        flush_ring()

                    e = slabhits[pl.ds(h * L, L)]
                    rem = jnp.minimum(cnt_s[3] - h * L, L)
                    valid = lanes < rem
                    kvec = (e >> 9) & (B - 1)
                    cvec = e & (SLABW - 1)
                    pos = oc + lanes
                    plsc.store_scatter(
                        oidx, [(pos >> 7) & 1, pos & 127], kvec, mask=valid
                    )
                    rows = pos & (RING - 1)
                    for d in range(D):
                        dsplat = jnp.full((L,), d, jnp.int32)
                        v = plsc.load_gather(buf, [dsplat, cvec], mask=valid)
                        v = jnp.maximum(v, v * NEG_SLOPE)
                        plsc.store_scatter(ring, [rows, dsplat], v, mask=valid)
                    cnt_s[1] = oc + rem

        @pl.loop(0, NPAIR)
        def _(u):
            t0 = 2 * u
            t1 = 2 * u + 1
            process_slab(t0, 0)
            fire_slab(t0 + 2, 0)
            process_slab(t1, 1)
            fire_slab(t1 + 2, 1)

        oc = cnt_s[1]
        npad = (-oc) & 127
        for i in range(128 // L):
            pos = oc + i * L + lanes
            valid = pos < oc + npad
            plsc.store_scatter(
                oidx, [(pos >> 7) & 1, pos & 127],
                jnp.full((L,), DUMMY, jnp.int32), mask=valid,
            )
        for _i in range(2):
            @pl.when(oc + npad - cnt_s[2] >= 128)
            def _():
                flush_ring()

    out = k(all_id, table_t, tail_t)
    return out[:B, :D]


# speedup vs baseline: 3.2967x; 1.1939x over previous
import functools

import jax
import jax.numpy as jnp
from jax import lax
from jax.experimental import pallas as pl
from jax.experimental.pallas import tpu as pltpu
from jax.experimental.pallas import tpu_sc as plsc

NC = 2
NS = 16
L = 16
NW = NC * NS

B = 16384
D = 64
V = 1000000

SLABW = 512
NSLAB = V // SLABW + 1
TAILS = V // SLABW
TAILW = V - TAILS * SLABW
NT = (NSLAB + NW - 1) // NW
NPAIR = (NT + 1) // 2

HITMAX = 768
BUCKMAX = 192
SSTRIP = 2048
NSTRIP = B // SSTRIP
OUTW = 128
RING = 256
DUMMY = B

NEG_SLOPE = 0.01


def kernel(all_id, table):
    table_t = table.T
    tail_t = jnp.pad(table_t[:, TAILS * SLABW:], ((0, 0), (0, 128 - TAILW)))
    mesh = plsc.VectorSubcoreMesh(core_axis_name="c", subcore_axis_name="s")

    @functools.partial(
        pl.kernel,
        out_type=jax.ShapeDtypeStruct((B + 8, OUTW), jnp.float32),
        mesh=mesh,
        compiler_params=pltpu.CompilerParams(needs_layout_passes=False),
        scratch_types=(
            [
                pltpu.VMEM((SSTRIP,), jnp.int32),
                pltpu.VMEM((SSTRIP,), jnp.int32),
                pltpu.VMEM((HITMAX + L,), jnp.int32),
                pltpu.VMEM((8, BUCKMAX + L), jnp.int32),
                pltpu.VMEM((128 + L,), jnp.int32),
                pltpu.VMEM((D, SLABW), jnp.float32),
                pltpu.VMEM((D, SLABW), jnp.float32),
                pltpu.VMEM((RING, OUTW), jnp.float32),
                pltpu.VMEM((4, 64), jnp.int32),
                pltpu.SMEM((16,), jnp.int32),
            ]
            + [pltpu.SemaphoreType.DMA for _ in range(5)]
        ),
    )
    def k(idx_hbm, table_hbm, tail_hbm, out_hbm, strip0, strip1, hitbuf,
          buckets, slabhits, slab0, slab1, ring, oidx, cnt_s,
          ss0, ss1, sg0, sg1, fsem):
        strips = (strip0, strip1)
        ssems = (ss0, ss1)
        slabs = (slab0, slab1)
        gsems = (sg0, sg1)

        wid = lax.axis_index("s") * NC + lax.axis_index("c")
        lanes = lax.iota(jnp.int32, L)
        lanes9 = lanes << 9

        cnt_s[0] = 0
        cnt_s[1] = 0
        cnt_s[2] = 0
        cnt_s[4] = 0

        @pl.loop(0, (HITMAX + L) // L)
        def _(g):
            hitbuf[pl.ds(g * L, L)] = jnp.full((L,), -1, jnp.int32)

        def fire_slab(t, tbuf):
            sid = wid + NW * t
            buf = slabs[tbuf]
            sem = gsems[tbuf]

            @pl.when(sid < TAILS)
            def _():
                pltpu.async_copy(
                    table_hbm.at[pl.ds(0, D), pl.ds(sid * SLABW, SLABW)],
                    buf, sem,
                )

            @pl.when(sid == TAILS)
            def _():
                pltpu.async_copy(
                    tail_hbm, buf.at[pl.ds(0, D), pl.ds(0, 128)], sem
                )

        def wait_slab(t, tbuf):
            sid = wid + NW * t
            buf = slabs[tbuf]
            sem = gsems[tbuf]

            @pl.when(sid < TAILS)
            def _():
                pltpu.make_async_copy(
                    table_hbm.at[pl.ds(0, D), pl.ds(0, SLABW)], buf, sem
                ).wait()

            @pl.when(sid == TAILS)
            def _():
                pltpu.make_async_copy(
                    tail_hbm, buf.at[pl.ds(0, D), pl.ds(0, 128)], sem
                ).wait()

        fire_slab(0, 0)
        fire_slab(1, 1)

        pltpu.async_copy(idx_hbm.at[pl.ds(0, SSTRIP)], strip0, ss0)
        for s in range(NSTRIP):
            if s + 1 < NSTRIP:
                pltpu.async_copy(
                    idx_hbm.at[pl.ds((s + 1) * SSTRIP, SSTRIP)],
                    strips[(s + 1) % 2], ssems[(s + 1) % 2],
                )
            pltpu.make_async_copy(
                idx_hbm.at[pl.ds(0, SSTRIP)], strips[s % 2], ssems[s % 2]
            ).wait()
            strip = strips[s % 2]

            @pl.loop(0, SSTRIP // L)
            def _(g, s=s, strip=strip):
                vec = strip[pl.ds(g * L, L)]
                mine = ((vec >> 9) & (NW - 1)) == wid
                pack = (
                    ((vec >> 14) << 23)
                    | (((s * SSTRIP + g * L) << 9) | lanes9)
                    | (vec & (SLABW - 1))
                )
                cnt = cnt_s[0]
                plsc.store_compressed(
                    hitbuf.at[pl.ds(cnt, L)], pack, mask=mine
                )
                cnt_s[0] = cnt + jnp.sum(jnp.where(mine, 1, 0))

        hcnt = cnt_s[0]
        for sb in range(8):
            cnt_s[8 + sb] = 0

        @pl.loop(0, (hcnt + L - 1) // L)
        def _(g):
            e = hitbuf[pl.ds(g * L, L)]
            sbv = e >> 26
            for sb in range(8):
                m = sbv == sb
                bc = cnt_s[8 + sb]
                plsc.store_compressed(
                    buckets.at[sb, pl.ds(bc, L)], e, mask=m
                )
                cnt_s[8 + sb] = bc + jnp.sum(jnp.where(m, 1, 0))

        def flush_ring():
            @pl.when(cnt_s[4] > 0)
            def _():
                pltpu.make_async_copy(
                    ring.at[pl.ds(0, 64)], out_hbm.at[oidx.at[0]], fsem
                ).wait()

            fl = cnt_s[2]
            half = (fl >> 6) & 3
            pltpu.async_copy(
                ring.at[pl.ds(fl & (RING - 1), 64)],
                out_hbm.at[oidx.at[half]], fsem,
            )
            cnt_s[2] = fl + 64
            cnt_s[4] = 1

        def process_slab(t, tbuf):
            sid = wid + NW * t
            buf = slabs[tbuf]

            @pl.when(sid <= TAILS)
            def _():
                sb = t >> 3
                bc = cnt_s[8 + sb]
                cnt_s[3] = 0

                @pl.loop(0, (bc + L - 1) // L)
                def _(g):
                    e = buckets[sb, pl.ds(g * L, L)]
                    m = ((e >> 23) == t) & (g * L + lanes < bc)
                    sc = cnt_s[3]
                    plsc.store_compressed(
                        slabhits.at[pl.ds(sc, L)], e & 0x7FFFFF, mask=m
                    )
                    cnt_s[3] = sc + jnp.sum(jnp.where(m, 1, 0))

                wait_slab(t, tbuf)
                sc = cnt_s[3]

                @pl.loop(0, (sc + L - 1) // L)
                def _(h):
                    oc = cnt_s[1]

                    @pl.when(oc - cnt_s[2] >= 64)
                    def _():
                        flush_ring()

                    e = slabhits[pl.ds(h * L, L)]
                    rem = jnp.minimum(cnt_s[3] - h * L, L)
                    valid = lanes < rem
                    kvec = (e >> 9) & (B - 1)
                    cvec = e & (SLABW - 1)
                    pos = oc + lanes
                    plsc.store_scatter(
                        oidx, [(pos >> 6) & 3, pos & 63], kvec, mask=valid
                    )
                    rows = pos & (RING - 1)
                    for d in range(D):
                        dsplat = jnp.full((L,), d, jnp.int32)
                        v = plsc.load_gather(buf, [dsplat, cvec], mask=valid)
                        v = jnp.maximum(v, v * NEG_SLOPE)
                        plsc.store_scatter(ring, [rows, dsplat], v, mask=valid)
                    cnt_s[1] = oc + rem

        @pl.loop(0, NPAIR)
        def _(u):
            t0 = 2 * u
            t1 = 2 * u + 1
            process_slab(t0, 0)
            fire_slab(t0 + 2, 0)
            process_slab(t1, 1)
            fire_slab(t1 + 2, 1)

        oc = cnt_s[1]
        npad = (-oc) & 63
        for i in range(64 // L):
            pos = oc + i * L + lanes
            valid = pos < oc + npad
            plsc.store_scatter(
                oidx, [(pos >> 6) & 3, pos & 63],
                jnp.full((L,), DUMMY, jnp.int32), mask=valid,
            )
        for _i in range(2):
            @pl.when(oc + npad - cnt_s[2] >= 64)
            def _():
                flush_ring()

        @pl.when(cnt_s[4] > 0)
        def _():
            pltpu.make_async_copy(
                ring.at[pl.ds(0, 64)], out_hbm.at[oidx.at[0]], fsem
            ).wait()

    out = k(all_id, table_t, tail_t)
    return out[:B, :D]


# speedup vs baseline: 3.3010x; 1.0013x over previous
import functools

import jax
import jax.numpy as jnp
from jax import lax
from jax.experimental import pallas as pl
from jax.experimental.pallas import tpu as pltpu
from jax.experimental.pallas import tpu_sc as plsc

NC = 2
NS = 16
L = 16
NW = NC * NS

B = 16384
D = 64
V = 1000000

SLABW = 512
NSLAB = V // SLABW + 1
TAILS = V // SLABW
TAILW = V - TAILS * SLABW
NT = (NSLAB + NW - 1) // NW
NPAIR = (NT + 1) // 2

HITMAX = 768
BUCKMAX = 192
SSTRIP = 2048
NSTRIP = B // SSTRIP
OUTW = 128
RING = 256
FCHUNK = 64
DUMMY = B

NEG_SLOPE = 0.01


def kernel(all_id, table):
    table_t = table.T
    tail_t = jnp.pad(table_t[:, TAILS * SLABW:], ((0, 0), (0, 128 - TAILW)))
    mesh = plsc.VectorSubcoreMesh(core_axis_name="c", subcore_axis_name="s")

    @functools.partial(
        pl.kernel,
        out_type=jax.ShapeDtypeStruct((B + 8, OUTW), jnp.float32),
        mesh=mesh,
        compiler_params=pltpu.CompilerParams(needs_layout_passes=False),
        scratch_types=(
            [
                pltpu.VMEM((SSTRIP,), jnp.int32),
                pltpu.VMEM((SSTRIP,), jnp.int32),
                pltpu.VMEM((HITMAX + L,), jnp.int32),
                pltpu.VMEM((8, BUCKMAX + L), jnp.int32),
                pltpu.VMEM((128 + L,), jnp.int32),
                pltpu.VMEM((D, SLABW), jnp.float32),
                pltpu.VMEM((D, SLABW), jnp.float32),
                pltpu.VMEM((RING, OUTW), jnp.float32),
                pltpu.VMEM((RING // FCHUNK, FCHUNK), jnp.int32),
                pltpu.SMEM((8,), jnp.int32),
            ]
            + [pltpu.SemaphoreType.DMA for _ in range(5)]
        ),
    )
    def k(idx_hbm, table_hbm, tail_hbm, out_hbm, strip0, strip1, hitbuf,
          buckets, slabhits, slab0, slab1, ring, oidx, bcnt_s,
          ss0, ss1, sg0, sg1, fsem):
        strips = (strip0, strip1)
        ssems = (ss0, ss1)
        slabs = (slab0, slab1)
        gsems = (sg0, sg1)

        wid = lax.axis_index("s") * NC + lax.axis_index("c")
        lanes = lax.iota(jnp.int32, L)
        lanes9 = lanes << 9

        def fire_slab(t, tbuf):
            sid = wid + NW * t
            buf = slabs[tbuf]
            sem = gsems[tbuf]

            @pl.when(sid < TAILS)
            def _():
                pltpu.async_copy(
                    table_hbm.at[pl.ds(0, D), pl.ds(sid * SLABW, SLABW)],
                    buf, sem,
                )

            @pl.when(sid == TAILS)
            def _():
                pltpu.async_copy(
                    tail_hbm, buf.at[pl.ds(0, D), pl.ds(0, 128)], sem
                )

        def wait_slab(t, tbuf):
            sid = wid + NW * t
            buf = slabs[tbuf]
            sem = gsems[tbuf]

            @pl.when(sid < TAILS)
            def _():
                pltpu.make_async_copy(
                    table_hbm.at[pl.ds(0, D), pl.ds(0, SLABW)], buf, sem
                ).wait()

            @pl.when(sid == TAILS)
            def _():
                pltpu.make_async_copy(
                    tail_hbm, buf.at[pl.ds(0, D), pl.ds(0, 128)], sem
                ).wait()

        fire_slab(0, 0)
        fire_slab(1, 1)

        hcnt = jnp.int32(0)
        pltpu.async_copy(idx_hbm.at[pl.ds(0, SSTRIP)], strip0, ss0)
        for s in range(NSTRIP):
            if s + 1 < NSTRIP:
                pltpu.async_copy(
                    idx_hbm.at[pl.ds((s + 1) * SSTRIP, SSTRIP)],
                    strips[(s + 1) % 2], ssems[(s + 1) % 2],
                )
            pltpu.make_async_copy(
                idx_hbm.at[pl.ds(0, SSTRIP)], strips[s % 2], ssems[s % 2]
            ).wait()
            strip = strips[s % 2]

            def scan_body(g, cnt, s=s, strip=strip):
                vec = strip[pl.ds(g * L, L)]
                mine = ((vec >> 9) & (NW - 1)) == wid
                pack = (
                    ((vec >> 14) << 23)
                    | (((s * SSTRIP + g * L) << 9) | lanes9)
                    | (vec & (SLABW - 1))
                )
                plsc.store_compressed(
                    hitbuf.at[pl.ds(cnt, L)], pack, mask=mine
                )
                return cnt + jnp.sum(jnp.where(mine, 1, 0))

            hcnt = lax.fori_loop(0, SSTRIP // L, scan_body, hcnt)

        def bucket_body(g, bcs):
            e = hitbuf[pl.ds(g * L, L)]
            sbv = e >> 26
            inb = g * L + lanes < hcnt
            out = []
            for sb in range(8):
                m = (sbv == sb) & inb
                plsc.store_compressed(
                    buckets.at[sb, pl.ds(bcs[sb], L)], e, mask=m
                )
                out.append(bcs[sb] + jnp.sum(jnp.where(m, 1, 0)))
            return tuple(out)

        bcs = lax.fori_loop(
            0, (hcnt + L - 1) // L, bucket_body,
            tuple(jnp.int32(0) for _ in range(8)),
        )
        for sb in range(8):
            bcnt_s[sb] = bcs[sb]

        def flush_ring(fl, pend):
            @pl.when(pend > 0)
            def _():
                pltpu.make_async_copy(
                    ring.at[pl.ds(0, FCHUNK)], out_hbm.at[oidx.at[0]], fsem
                ).wait()

            half = (fl >> 6) & (RING // FCHUNK - 1)
            pltpu.async_copy(
                ring.at[pl.ds(fl & (RING - 1), FCHUNK)],
                out_hbm.at[oidx.at[half]], fsem,
            )
            return fl + FCHUNK, jnp.int32(1)

        def process_slab(t, tbuf, carry):
            oc, fl, pend = carry
            buf = slabs[tbuf]
            sb = t >> 3
            bc = bcnt_s[sb]

            def filter_body(g, sc):
                e = buckets[sb, pl.ds(g * L, L)]
                m = ((e >> 23) == t) & (g * L + lanes < bc)
                plsc.store_compressed(
                    slabhits.at[pl.ds(sc, L)], e & 0x7FFFFF, mask=m
                )
                return sc + jnp.sum(jnp.where(m, 1, 0))

            sc = lax.fori_loop(0, (bc + L - 1) // L, filter_body,
                               jnp.int32(0))
            wait_slab(t, tbuf)

            def extract_body(h, carry):
                oc, fl, pend = carry
                do_flush = oc - fl >= FCHUNK
                fl2, pend2 = lax.cond(
                    do_flush, flush_ring, lambda a, b: (a, b), fl, pend
                )
                e = slabhits[pl.ds(h * L, L)]
                rem = jnp.minimum(sc - h * L, L)
                valid = lanes < rem
                kvec = (e >> 9) & (B - 1)
                cvec = e & (SLABW - 1)
                pos = oc + lanes
                plsc.store_scatter(
                    oidx, [(pos >> 6) & (RING // FCHUNK - 1),
                           pos & (FCHUNK - 1)], kvec, mask=valid,
                )
                rows = pos & (RING - 1)
                for d in range(D):
                    dsplat = jnp.full((L,), d, jnp.int32)
                    v = plsc.load_gather(buf, [dsplat, cvec], mask=valid)
                    v = jnp.maximum(v, v * NEG_SLOPE)
                    plsc.store_scatter(ring, [rows, dsplat], v, mask=valid)
                return oc + rem, fl2, pend2

            return lax.fori_loop(0, (sc + L - 1) // L, extract_body,
                                 (oc, fl, pend))

        def round_body(u, carry):
            t0 = 2 * u
            carry = process_slab(t0, 0, carry)
            fire_slab(t0 + 2, 0)
            carry = process_slab(t0 + 1, 1, carry)
            fire_slab(t0 + 3, 1)
            return carry

        oc, fl, pend = lax.fori_loop(
            0, NPAIR, round_body,
            (jnp.int32(0), jnp.int32(0), jnp.int32(0)),
        )

        npad = (-oc) & (FCHUNK - 1)
        for i in range(FCHUNK // L):
            pos = oc + i * L + lanes
            valid = pos < oc + npad
            plsc.store_scatter(
                oidx, [(pos >> 6) & (RING // FCHUNK - 1), pos & (FCHUNK - 1)],
                jnp.full((L,), DUMMY, jnp.int32), mask=valid,
            )
        for _i in range(2):
            fl, pend = lax.cond(
                oc + npad - fl >= FCHUNK, flush_ring,
                lambda a, b: (a, b), fl, pend,
            )

        @pl.when(pend > 0)
        def _():
            pltpu.make_async_copy(
                ring.at[pl.ds(0, FCHUNK)], out_hbm.at[oidx.at[0]], fsem
            ).wait()

    out = k(all_id, table_t, tail_t)
    return out[:B, :D]


# speedup vs baseline: 3.9024x; 1.1822x over previous
import functools

import jax
import jax.numpy as jnp
from jax import lax
from jax.experimental import pallas as pl
from jax.experimental.pallas import tpu as pltpu
from jax.experimental.pallas import tpu_sc as plsc

NC = 2
NS = 16
L = 16
NW = NC * NS

B = 16384
D = 64
V = 1000000

SLABW = 512
NSLAB = V // SLABW + 1
TAILS = V // SLABW
TAILW = V - TAILS * SLABW
NT = (NSLAB + NW - 1) // NW
NBUF = 3
NROUND = (NT + NBUF - 1) // NBUF

HITMAX = 768
BUCKMAX = 192
SSTRIP = 1024
NSTRIP = B // SSTRIP
OUTW = 128
RING = 128
FCHUNK = 32
DUMMY = B

NEG_SLOPE = 0.01


def kernel(all_id, table):
    table_t = table.T
    tail_t = jnp.pad(table_t[:, TAILS * SLABW:], ((0, 0), (0, 128 - TAILW)))
    mesh = plsc.VectorSubcoreMesh(core_axis_name="c", subcore_axis_name="s")

    @functools.partial(
        pl.kernel,
        out_type=jax.ShapeDtypeStruct((B + 8, OUTW), jnp.float32),
        mesh=mesh,
        compiler_params=pltpu.CompilerParams(needs_layout_passes=False),
        scratch_types=(
            [
                pltpu.VMEM((SSTRIP,), jnp.int32),
                pltpu.VMEM((SSTRIP,), jnp.int32),
                pltpu.VMEM((HITMAX + L,), jnp.int32),
                pltpu.VMEM((8, BUCKMAX + L), jnp.int32),
                pltpu.VMEM((128 + L,), jnp.int32),
                pltpu.VMEM((D, SLABW), jnp.float32),
                pltpu.VMEM((D, SLABW), jnp.float32),
                pltpu.VMEM((D, SLABW), jnp.float32),
                pltpu.VMEM((RING, OUTW), jnp.float32),
                pltpu.VMEM((RING // FCHUNK, FCHUNK), jnp.int32),
                pltpu.SMEM((8,), jnp.int32),
            ]
            + [pltpu.SemaphoreType.DMA for _ in range(6)]
        ),
    )
    def k(idx_hbm, table_hbm, tail_hbm, out_hbm, strip0, strip1, hitbuf,
          buckets, slabhits, slab0, slab1, slab2, ring, oidx, bcnt_s,
          ss0, ss1, sg0, sg1, sg2, fsem):
        strips = (strip0, strip1)
        ssems = (ss0, ss1)
        slabs = (slab0, slab1, slab2)
        gsems = (sg0, sg1, sg2)

        wid = lax.axis_index("s") * NC + lax.axis_index("c")
        lanes = lax.iota(jnp.int32, L)
        lanes9 = lanes << 9

        def fire_slab(t, tbuf):
            sid = wid + NW * t
            buf = slabs[tbuf]
            sem = gsems[tbuf]

            @pl.when(sid < TAILS)
            def _():
                pltpu.async_copy(
                    table_hbm.at[pl.ds(0, D), pl.ds(sid * SLABW, SLABW)],
                    buf, sem,
                )

            @pl.when(sid == TAILS)
            def _():
                pltpu.async_copy(
                    tail_hbm, buf.at[pl.ds(0, D), pl.ds(0, 128)], sem
                )

        def wait_slab(t, tbuf):
            sid = wid + NW * t
            buf = slabs[tbuf]
            sem = gsems[tbuf]

            @pl.when(sid < TAILS)
            def _():
                pltpu.make_async_copy(
                    table_hbm.at[pl.ds(0, D), pl.ds(0, SLABW)], buf, sem
                ).wait()

            @pl.when(sid == TAILS)
            def _():
                pltpu.make_async_copy(
                    tail_hbm, buf.at[pl.ds(0, D), pl.ds(0, 128)], sem
                ).wait()

        fire_slab(0, 0)
        fire_slab(1, 1)
        fire_slab(2, 2)

        hcnt = jnp.int32(0)
        pltpu.async_copy(idx_hbm.at[pl.ds(0, SSTRIP)], strip0, ss0)
        for s in range(NSTRIP):
            if s + 1 < NSTRIP:
                pltpu.async_copy(
                    idx_hbm.at[pl.ds((s + 1) * SSTRIP, SSTRIP)],
                    strips[(s + 1) % 2], ssems[(s + 1) % 2],
                )
            pltpu.make_async_copy(
                idx_hbm.at[pl.ds(0, SSTRIP)], strips[s % 2], ssems[s % 2]
            ).wait()
            strip = strips[s % 2]

            def scan_body(g, cnt, s=s, strip=strip):
                vec = strip[pl.ds(g * L, L)]
                mine = ((vec >> 9) & (NW - 1)) == wid
                pack = (
                    ((vec >> 14) << 23)
                    | (((s * SSTRIP + g * L) << 9) | lanes9)
                    | (vec & (SLABW - 1))
                )
                plsc.store_compressed(
                    hitbuf.at[pl.ds(cnt, L)], pack, mask=mine
                )
                return cnt + jnp.sum(jnp.where(mine, 1, 0))

            hcnt = lax.fori_loop(0, SSTRIP // L, scan_body, hcnt)

        def bucket_body(g, bcs):
            e = hitbuf[pl.ds(g * L, L)]
            sbv = e >> 26
            inb = g * L + lanes < hcnt
            out = []
            for sb in range(8):
                m = (sbv == sb) & inb
                plsc.store_compressed(
                    buckets.at[sb, pl.ds(bcs[sb], L)], e, mask=m
                )
                out.append(bcs[sb] + jnp.sum(jnp.where(m, 1, 0)))
            return tuple(out)

        bcs = lax.fori_loop(
            0, (hcnt + L - 1) // L, bucket_body,
            tuple(jnp.int32(0) for _ in range(8)),
        )
        for sb in range(8):
            bcnt_s[sb] = bcs[sb]

        def flush_ring(fl, pend):
            @pl.when(pend > 0)
            def _():
                pltpu.make_async_copy(
                    ring.at[pl.ds(0, FCHUNK)], out_hbm.at[oidx.at[0]], fsem
                ).wait()

            half = (fl >> 5) & (RING // FCHUNK - 1)
            pltpu.async_copy(
                ring.at[pl.ds(fl & (RING - 1), FCHUNK)],
                out_hbm.at[oidx.at[half]], fsem,
            )
            return fl + FCHUNK, jnp.int32(1)

        def process_slab(t, tbuf, carry):
            oc, fl, pend = carry
            buf = slabs[tbuf]
            sb = t >> 3
            bc = bcnt_s[sb]

            def filter_body(g, sc):
                e = buckets[sb, pl.ds(g * L, L)]
                m = ((e >> 23) == t) & (g * L + lanes < bc)
                plsc.store_compressed(
                    slabhits.at[pl.ds(sc, L)], e & 0x7FFFFF, mask=m
                )
                return sc + jnp.sum(jnp.where(m, 1, 0))

            sc = lax.fori_loop(0, (bc + L - 1) // L, filter_body,
                               jnp.int32(0))
            wait_slab(t, tbuf)

            def extract_body(h, carry):
                oc, fl, pend = carry
                do_flush = oc - fl >= FCHUNK
                fl2, pend2 = lax.cond(
                    do_flush, flush_ring, lambda a, b: (a, b), fl, pend
                )
                e = slabhits[pl.ds(h * L, L)]
                rem = jnp.minimum(sc - h * L, L)
                valid = lanes < rem
                kvec = (e >> 9) & (B - 1)
                cvec = e & (SLABW - 1)
                pos = oc + lanes
                plsc.store_scatter(
                    oidx, [(pos >> 5) & (RING // FCHUNK - 1),
                           pos & (FCHUNK - 1)], kvec, mask=valid,
                )
                rows = pos & (RING - 1)
                for d in range(D):
                    dsplat = jnp.full((L,), d, jnp.int32)
                    v = plsc.load_gather(buf, [dsplat, cvec], mask=valid)
                    v = jnp.maximum(v, v * NEG_SLOPE)
                    plsc.store_scatter(ring, [rows, dsplat], v, mask=valid)
                return oc + rem, fl2, pend2

            return lax.fori_loop(0, (sc + L - 1) // L, extract_body,
                                 (oc, fl, pend))

        def round_body(u, carry):
            t0 = NBUF * u
            for b in range(NBUF):
                carry = process_slab(t0 + b, b, carry)
                fire_slab(t0 + b + NBUF, b)
            return carry

        oc, fl, pend = lax.fori_loop(
            0, NROUND, round_body,
            (jnp.int32(0), jnp.int32(0), jnp.int32(0)),
        )

        npad = (-oc) & (FCHUNK - 1)
        for i in range(FCHUNK // L):
            pos = oc + i * L + lanes
            valid = pos < oc + npad
            plsc.store_scatter(
                oidx, [(pos >> 5) & (RING // FCHUNK - 1), pos & (FCHUNK - 1)],
                jnp.full((L,), DUMMY, jnp.int32), mask=valid,
            )
        for _i in range(2):
            fl, pend = lax.cond(
                oc + npad - fl >= FCHUNK, flush_ring,
                lambda a, b: (a, b), fl, pend,
            )

        @pl.when(pend > 0)
        def _():
            pltpu.make_async_copy(
                ring.at[pl.ds(0, FCHUNK)], out_hbm.at[oidx.at[0]], fsem
            ).wait()

    out = k(all_id, table_t, tail_t)
    return out[:B, :D]


# speedup vs baseline: 3.9068x; 1.0011x over previous
import functools

import jax
import jax.numpy as jnp
from jax import lax
from jax.experimental import pallas as pl
from jax.experimental.pallas import tpu as pltpu
from jax.experimental.pallas import tpu_sc as plsc

NC = 2
NS = 16
L = 16
NW = NC * NS

B = 16384
D = 64
V = 1000000

SLABW = 512
NSLAB = V // SLABW + 1
TAILS = V // SLABW
TAILW = V - TAILS * SLABW
NT = (NSLAB + NW - 1) // NW
NBUF = 3
NROUND = (NT + NBUF - 1) // NBUF

HITMAX = 768
BUCKMAX = 192
SSTRIP = 1024
NSTRIP = B // SSTRIP
OUTW = 128
RING = 128
FCHUNK = 32
DUMMY = B

NEG_SLOPE = 0.01


def kernel(all_id, table):
    table_t = table.T
    tail_t = jnp.pad(table_t[:, TAILS * SLABW:], ((0, 0), (0, 128 - TAILW)))
    mesh = plsc.VectorSubcoreMesh(core_axis_name="c", subcore_axis_name="s")

    @functools.partial(
        pl.kernel,
        out_type=jax.ShapeDtypeStruct((B + 8, OUTW), jnp.float32),
        mesh=mesh,
        compiler_params=pltpu.CompilerParams(needs_layout_passes=False),
        scratch_types=(
            [
                pltpu.VMEM((SSTRIP,), jnp.int32),
                pltpu.VMEM((SSTRIP,), jnp.int32),
                pltpu.VMEM((HITMAX + L,), jnp.int32),
                pltpu.VMEM((8, BUCKMAX + L), jnp.int32),
                pltpu.VMEM((128 + L,), jnp.int32),
                pltpu.VMEM((D, SLABW), jnp.float32),
                pltpu.VMEM((D, SLABW), jnp.float32),
                pltpu.VMEM((D, SLABW), jnp.float32),
                pltpu.VMEM((RING, OUTW), jnp.float32),
                pltpu.VMEM((RING // FCHUNK, FCHUNK), jnp.int32),
                pltpu.SMEM((8,), jnp.int32),
            ]
            + [pltpu.SemaphoreType.DMA for _ in range(6)]
        ),
    )
    def k(idx_hbm, table_hbm, tail_hbm, out_hbm, strip0, strip1, hitbuf,
          buckets, slabhits, slab0, slab1, slab2, ring, oidx, bcnt_s,
          ss0, ss1, sg0, sg1, sg2, fsem):
        strips = (strip0, strip1)
        ssems = (ss0, ss1)
        slabs = (slab0, slab1, slab2)
        gsems = (sg0, sg1, sg2)

        wid = lax.axis_index("s") * NC + lax.axis_index("c")
        lanes = lax.iota(jnp.int32, L)
        lanes9 = lanes << 9

        def fire_slab(t, tbuf):
            sid = wid + NW * t
            buf = slabs[tbuf]
            sem = gsems[tbuf]

            @pl.when(sid < TAILS)
            def _():
                for r in range(D // 8):
                    pltpu.async_copy(
                        table_hbm.at[pl.ds(8 * r, 8),
                                     pl.ds(sid * SLABW, SLABW)],
                        buf.at[pl.ds(8 * r, 8), pl.ds(0, SLABW)], sem,
                    )

            @pl.when(sid == TAILS)
            def _():
                pltpu.async_copy(
                    tail_hbm, buf.at[pl.ds(0, D), pl.ds(0, 128)], sem
                )

        def wait_slab(t, tbuf):
            sid = wid + NW * t
            buf = slabs[tbuf]
            sem = gsems[tbuf]

            @pl.when(sid < TAILS)
            def _():
                pltpu.make_async_copy(
                    table_hbm.at[pl.ds(0, D), pl.ds(0, SLABW)], buf, sem
                ).wait()

            @pl.when(sid == TAILS)
            def _():
                pltpu.make_async_copy(
                    tail_hbm, buf.at[pl.ds(0, D), pl.ds(0, 128)], sem
                ).wait()

        fire_slab(0, 0)
        fire_slab(1, 1)
        fire_slab(2, 2)

        hcnt = jnp.int32(0)
        pltpu.async_copy(idx_hbm.at[pl.ds(0, SSTRIP)], strip0, ss0)
        for s in range(NSTRIP):
            if s + 1 < NSTRIP:
                pltpu.async_copy(
                    idx_hbm.at[pl.ds((s + 1) * SSTRIP, SSTRIP)],
                    strips[(s + 1) % 2], ssems[(s + 1) % 2],
                )
            pltpu.make_async_copy(
                idx_hbm.at[pl.ds(0, SSTRIP)], strips[s % 2], ssems[s % 2]
            ).wait()
            strip = strips[s % 2]

            def scan_body(g, cnt, s=s, strip=strip):
                vec = strip[pl.ds(g * L, L)]
                mine = ((vec >> 9) & (NW - 1)) == wid
                pack = (
                    ((vec >> 14) << 23)
                    | (((s * SSTRIP + g * L) << 9) | lanes9)
                    | (vec & (SLABW - 1))
                )
                plsc.store_compressed(
                    hitbuf.at[pl.ds(cnt, L)], pack, mask=mine
                )
                return cnt + jnp.sum(jnp.where(mine, 1, 0))

            hcnt = lax.fori_loop(0, SSTRIP // L, scan_body, hcnt)

        def bucket_body(g, bcs):
            e = hitbuf[pl.ds(g * L, L)]
            sbv = e >> 26
            inb = g * L + lanes < hcnt
            out = []
            for sb in range(8):
                m = (sbv == sb) & inb
                plsc.store_compressed(
                    buckets.at[sb, pl.ds(bcs[sb], L)], e, mask=m
                )
                out.append(bcs[sb] + jnp.sum(jnp.where(m, 1, 0)))
            return tuple(out)

        bcs = lax.fori_loop(
            0, (hcnt + L - 1) // L, bucket_body,
            tuple(jnp.int32(0) for _ in range(8)),
        )
        for sb in range(8):
            bcnt_s[sb] = bcs[sb]

        def flush_ring(fl, pend):
            @pl.when(pend > 0)
            def _():
                pltpu.make_async_copy(
                    ring.at[pl.ds(0, FCHUNK)], out_hbm.at[oidx.at[0]], fsem
                ).wait()

            half = (fl >> 5) & (RING // FCHUNK - 1)
            pltpu.async_copy(
                ring.at[pl.ds(fl & (RING - 1), FCHUNK)],
                out_hbm.at[oidx.at[half]], fsem,
            )
            return fl + FCHUNK, jnp.int32(1)

        def process_slab(t, tbuf, carry):
            oc, fl, pend = carry
            buf = slabs[tbuf]
            sb = t >> 3
            bc = bcnt_s[sb]

            def filter_body(g, sc):
                e = buckets[sb, pl.ds(g * L, L)]
                m = ((e >> 23) == t) & (g * L + lanes < bc)
                plsc.store_compressed(
                    slabhits.at[pl.ds(sc, L)], e & 0x7FFFFF, mask=m
                )
                return sc + jnp.sum(jnp.where(m, 1, 0))

            sc = lax.fori_loop(0, (bc + L - 1) // L, filter_body,
                               jnp.int32(0))
            wait_slab(t, tbuf)

            def extract_body(h, carry):
                oc, fl, pend = carry
                do_flush = oc - fl >= FCHUNK
                fl2, pend2 = lax.cond(
                    do_flush, flush_ring, lambda a, b: (a, b), fl, pend
                )
                e = slabhits[pl.ds(h * L, L)]
                rem = jnp.minimum(sc - h * L, L)
                valid = lanes < rem
                kvec = (e >> 9) & (B - 1)
                cvec = e & (SLABW - 1)
                pos = oc + lanes
                plsc.store_scatter(
                    oidx, [(pos >> 5) & (RING // FCHUNK - 1),
                           pos & (FCHUNK - 1)], kvec, mask=valid,
                )
                rows = pos & (RING - 1)
                for d in range(D):
                    dsplat = jnp.full((L,), d, jnp.int32)
                    v = plsc.load_gather(buf, [dsplat, cvec], mask=valid)
                    v = jnp.maximum(v, v * NEG_SLOPE)
                    plsc.store_scatter(ring, [rows, dsplat], v, mask=valid)
                return oc + rem, fl2, pend2

            return lax.fori_loop(0, (sc + L - 1) // L, extract_body,
                                 (oc, fl, pend))

        def round_body(u, carry):
            t0 = NBUF * u
            for b in range(NBUF):
                carry = process_slab(t0 + b, b, carry)
                fire_slab(t0 + b + NBUF, b)
            return carry

        oc, fl, pend = lax.fori_loop(
            0, NROUND, round_body,
            (jnp.int32(0), jnp.int32(0), jnp.int32(0)),
        )

        npad = (-oc) & (FCHUNK - 1)
        for i in range(FCHUNK // L):
            pos = oc + i * L + lanes
            valid = pos < oc + npad
            plsc.store_scatter(
                oidx, [(pos >> 5) & (RING // FCHUNK - 1), pos & (FCHUNK - 1)],
                jnp.full((L,), DUMMY, jnp.int32), mask=valid,
            )
        for _i in range(2):
            fl, pend = lax.cond(
                oc + npad - fl >= FCHUNK, flush_ring,
                lambda a, b: (a, b), fl, pend,
            )

        @pl.when(pend > 0)
        def _():
            pltpu.make_async_copy(
                ring.at[pl.ds(0, FCHUNK)], out_hbm.at[oidx.at[0]], fsem
            ).wait()

    out = k(all_id, table_t, tail_t)
    return out[:B, :D]
